# baseline (speedup 1.0000x reference)
import functools

import jax
import jax.numpy as jnp
from jax import lax
from jax.experimental import pallas as pl
from jax.experimental.pallas import tpu as pltpu
from jax.experimental.pallas import tpu_sc as plsc


def _sc_body(rows, cols, k, rpt, nc, x_hbm, idx_hbm, out_hbm, idx_v, buf_v,
             sem):
    wid = lax.axis_index("s") * nc + lax.axis_index("c")
    base = wid * rpt
    pltpu.sync_copy(idx_hbm, idx_v)

    idx_vecs = [idx_v[pl.ds(g * 16, 16)] for g in range(k // 16)]

    for j in range(k):
        col = idx_vecs[j // 16][j % 16]
        pltpu.make_async_copy(
            x_hbm.at[pl.ds(base, rpt), pl.ds(col, 1)],
            buf_v.at[:, pl.ds(j, 1)],
            sem,
        ).start()

    pltpu.make_async_copy(
        x_hbm.at[pl.ds(0, rpt), pl.ds(0, k)], buf_v, sem).wait()

    pltpu.sync_copy(buf_v, out_hbm.at[pl.ds(base, rpt), :])


def kernel(x, indices):
    b, s, c = x.shape
    k = indices.shape[0]
    rows = b * s
    x2 = x.reshape(rows, c)

    info = plsc.get_sparse_core_info()
    nc, ns = info.num_cores, info.num_subcores
    nw = nc * ns
    rpt = rows // nw

    mesh = plsc.VectorSubcoreMesh(core_axis_name="c", subcore_axis_name="s")
    sc_call = pl.kernel(
        functools.partial(_sc_body, rows, c, k, rpt, nc),
        mesh=mesh,
        out_type=jax.ShapeDtypeStruct((rows, k), jnp.float32),
        scratch_types=[
            pltpu.VMEM((k,), jnp.int32),
            pltpu.VMEM((rpt, k), jnp.float32),
            pltpu.SemaphoreType.DMA,
        ],
        compiler_params=pltpu.CompilerParams(use_tc_tiling_on_sc=False),
    )
    out = sc_call(x2, indices)
    return out.reshape(b, s, k)


# speedup vs baseline: 3.7362x; 3.7362x over previous
import jax
import jax.numpy as jnp
from jax.experimental import pallas as pl
from jax.experimental.pallas import tpu as pltpu


def _body(idx_ref, x_ref, o_ref):
    c = x_ref.shape[1]
    k = o_ref.shape[1]
    col = jax.lax.broadcasted_iota(jnp.int32, (c, k), 0)
    onehot = (col == idx_ref[0, :][None, :]).astype(jnp.float32)
    o_ref[...] = jnp.dot(x_ref[...], onehot, preferred_element_type=jnp.float32)


def kernel(x, indices):
    b, s, c = x.shape
    k = indices.shape[0]
    rows = b * s
    x2 = x.reshape(rows, c)
    blk = 512
    out = pl.pallas_call(
        _body,
        grid=(rows // blk,),
        in_specs=[
            pl.BlockSpec((1, k), lambda i: (0, 0)),
            pl.BlockSpec((blk, c), lambda i: (i, 0)),
        ],
        out_specs=pl.BlockSpec((blk, k), lambda i: (i, 0)),
        out_shape=jax.ShapeDtypeStruct((rows, k), jnp.float32),
    )(indices.reshape(1, k), x2)
    return out.reshape(b, s, k)
